# baseline (speedup 1.0000x reference)
import jax, jax.numpy as jnp
from jax.experimental import pallas as pl

N = 10000

def _id_body(x_ref, o_ref):
    o_ref[...] = x_ref[...]

def _pl_id(x):
    return pl.pallas_call(_id_body, out_shape=jax.ShapeDtypeStruct(x.shape, x.dtype))(x)

def kernel(x, edge_index, W_conv, W_res, b_res, bn_gamma, bn_beta, W_ih, W_hh, b_lstm, ln_gamma, ln_beta):
    src = edge_index[0]
    dst = edge_index[1]
    h = x
    for l in range(2):
        score = jnp.sum(h[src] * h[dst], axis=-1)
        smax = jax.ops.segment_max(score, dst, num_segments=N)
        smax = jnp.where(jnp.isfinite(smax), smax, 0.0)
        ex = jnp.exp(score - smax[dst])
        denom = jax.ops.segment_sum(ex, dst, num_segments=N)
        a = ex / (denom[dst] + 1e-9)
        ft = h @ W_conv[l]
        msg = ft[src] * a[:, None]
        agg = jax.ops.segment_sum(msg, dst, num_segments=N)
        res = jax.nn.relu(h @ W_res[l] + b_res[l])
        h = agg + res
        mean = h.mean(axis=0)
        var = h.var(axis=0)
        h = bn_gamma[l] * (h - mean) / jnp.sqrt(var + 1e-5) + bn_beta[l]
    q_star = jnp.zeros((1, 256), dtype=h.dtype)
    hs = jnp.zeros((1, 128), dtype=h.dtype)
    cs = jnp.zeros((1, 128), dtype=h.dtype)
    for _ in range(3):
        gates = q_star @ W_ih + hs @ W_hh + b_lstm
        i_g, f_g, g_g, o_g = jnp.split(gates, 4, axis=-1)
        cs = jax.nn.sigmoid(f_g) * cs + jax.nn.sigmoid(i_g) * jnp.tanh(g_g)
        hs = jax.nn.sigmoid(o_g) * jnp.tanh(cs)
        q = hs
        e = h @ q.T
        alpha = jax.nn.softmax(e, axis=0)
        r = jnp.sum(alpha * h, axis=0, keepdims=True)
        q_star = jnp.concatenate([q, r], axis=-1)
    mean = q_star.mean(axis=-1, keepdims=True)
    var = q_star.var(axis=-1, keepdims=True)
    out = ln_gamma * (q_star - mean) / jnp.sqrt(var + 1e-5) + ln_beta
    return _pl_id(out)


# speedup vs baseline: 1.0832x; 1.0832x over previous
import jax
import jax.numpy as jnp
from jax import lax
from jax.experimental import pallas as pl
from jax.experimental.pallas import tpu as pltpu
from jax.experimental.pallas import tpu_sc as plsc

N = 10000
E = 320000
D = 128
NPAD = 10240
NC, NS = 2, 16
NW = NC * NS
EPW = E // NW
SLW = NPAD // NW
SENT = -3.0e38

_MESH = plsc.VectorSubcoreMesh(core_axis_name="c", subcore_axis_name="s")
_SC_PARAMS = pltpu.CompilerParams(needs_layout_passes=False)


def _lmax_body(score_hbm, dst_hbm, lmax_hbm, dstb, sb, lmax):
    c = lax.axis_index("c")
    s = lax.axis_index("s")
    wid = c * NS + s
    start = wid * EPW

    def init_i(i, _):
        lmax[pl.ds(i * 16, 16)] = jnp.full((16,), SENT, jnp.float32)
        return 0
    lax.fori_loop(0, NPAD // 16, init_i, 0)

    pltpu.sync_copy(score_hbm.at[pl.ds(start, EPW)], sb)
    pltpu.sync_copy(dst_hbm.at[pl.ds(start, EPW)], dstb)

    def group(g, _):
        dvec = dstb[pl.ds(g * 16, 16)]
        svec = sb[pl.ds(g * 16, 16)]
        cur0 = plsc.load_gather(lmax, [dvec])
        plsc.store_scatter(lmax, [dvec], jnp.maximum(cur0, svec))

        def cond(k):
            return k > 0

        def body(k):
            cur = plsc.load_gather(lmax, [dvec])
            m = svec > cur
            plsc.store_scatter(lmax, [dvec], svec, mask=m)
            return jnp.sum(m.astype(jnp.int32))
        lax.while_loop(cond, body, jnp.int32(1))
        return 0
    lax.fori_loop(0, EPW // 16, group, 0)

    pltpu.sync_copy(lmax, lmax_hbm.at[pl.ds(wid * NPAD, NPAD)])


def _sc_lmax(score, dst):
    return pl.kernel(
        _lmax_body,
        out_type=jax.ShapeDtypeStruct((NW * NPAD,), jnp.float32),
        mesh=_MESH,
        compiler_params=_SC_PARAMS,
        scratch_types=[
            pltpu.VMEM((EPW,), jnp.int32),
            pltpu.VMEM((EPW,), jnp.float32),
            pltpu.VMEM((NPAD,), jnp.float32),
        ],
    )(score, dst)


def _smax_body(lmax_hbm, smax_hbm, lmbuf, outb):
    c = lax.axis_index("c")
    s = lax.axis_index("s")
    wid = c * NS + s

    for w in range(NW):
        pltpu.sync_copy(lmax_hbm.at[pl.ds(w * NPAD + wid * SLW, SLW)],
                        lmbuf.at[pl.ds(w * SLW, SLW)])

    def red_i(i, _):
        acc = lmbuf[pl.ds(i * 16, 16)]
        for w in range(1, NW):
            acc = jnp.maximum(acc, lmbuf[pl.ds(w * SLW + i * 16, 16)])
        acc = jnp.where(acc < -1e30, 0.0, acc)
        outb[pl.ds(i * 16, 16)] = acc
        return 0
    lax.fori_loop(0, SLW // 16, red_i, 0)

    pltpu.sync_copy(outb, smax_hbm.at[pl.ds(wid * SLW, SLW)])


def _sc_smax(lmax):
    return pl.kernel(
        _smax_body,
        out_type=jax.ShapeDtypeStruct((NPAD,), jnp.float32),
        mesh=_MESH,
        compiler_params=_SC_PARAMS,
        scratch_types=[
            pltpu.VMEM((NW * SLW,), jnp.float32),
            pltpu.VMEM((SLW,), jnp.float32),
        ],
    )(lmax)


def _segment_max(score, dst):
    return _sc_smax(_sc_lmax(score, dst))[:N]


def kernel(x, edge_index, W_conv, W_res, b_res, bn_gamma, bn_beta,
           W_ih, W_hh, b_lstm, ln_gamma, ln_beta):
    src = edge_index[0]
    dst = edge_index[1]
    h = x
    for l in range(2):
        score = jnp.sum(h[src] * h[dst], axis=-1)
        smax = _segment_max(score, dst)
        ex = jnp.exp(score - smax[dst])
        denom = jax.ops.segment_sum(ex, dst, num_segments=N)
        a = ex / (denom[dst] + 1e-9)
        ft = h @ W_conv[l]
        msg = ft[src] * a[:, None]
        agg = jax.ops.segment_sum(msg, dst, num_segments=N)
        res = jax.nn.relu(h @ W_res[l] + b_res[l])
        h = agg + res
        mean = h.mean(axis=0)
        var = h.var(axis=0)
        h = bn_gamma[l] * (h - mean) / jnp.sqrt(var + 1e-5) + bn_beta[l]
    q_star = jnp.zeros((1, 2 * D), dtype=h.dtype)
    hs = jnp.zeros((1, D), dtype=h.dtype)
    cs = jnp.zeros((1, D), dtype=h.dtype)
    for _ in range(3):
        gates = q_star @ W_ih + hs @ W_hh + b_lstm
        i_g, f_g, g_g, o_g = jnp.split(gates, 4, axis=-1)
        cs = jax.nn.sigmoid(f_g) * cs + jax.nn.sigmoid(i_g) * jnp.tanh(g_g)
        hs = jax.nn.sigmoid(o_g) * jnp.tanh(cs)
        q = hs
        e = h @ q.T
        alpha = jax.nn.softmax(e, axis=0)
        r = jnp.sum(alpha * h, axis=0, keepdims=True)
        q_star = jnp.concatenate([q, r], axis=-1)
    mean = q_star.mean(axis=-1, keepdims=True)
    var = q_star.var(axis=-1, keepdims=True)
    out = ln_gamma * (q_star - mean) / jnp.sqrt(var + 1e-5) + ln_beta
    return out
